# Initial kernel scaffold; baseline (speedup 1.0000x reference)
import jax
import jax.numpy as jnp
from jax.experimental import pallas as pl


def kernel(graph_x, edge_index, batch, W1, b1, gamma1, beta1, W2, b2, gamma2, beta2, Wh, bh):
    raise NotImplementedError("write your pallas kernel here")


# speedup vs baseline: 6.5197x; 6.5197x over previous
import jax
import jax.numpy as jnp
from jax import lax
from jax.experimental import pallas as pl
from jax.experimental.pallas import tpu as pltpu
from jax.experimental.pallas import tpu_sc as plsc

_NC = 2
_NS = 16
_CH = 128
_EPS = 1e-5


def _round_up(x, m):
    return (x + m - 1) // m * m


_DW = 128


def _sc_degree(dst2d, nacc, interpret=False):
    rows_total = dst2d.shape[0]
    rpt = rows_total // (_NC * _NS)
    spt = nacc // _NS
    mesh = plsc.VectorSubcoreMesh(core_axis_name="c", subcore_axis_name="s")

    def body(dst_hbm, ones_hbm, z_hbm, out_hbm, dst_s, ones_v, acc):
        cid = lax.axis_index("c")
        sid = lax.axis_index("s")
        base = (cid * _NS + sid) * rpt
        pltpu.sync_copy(dst_hbm.at[pl.ds(base, rpt)], dst_s)
        pltpu.sync_copy(ones_hbm, ones_v)
        pltpu.sync_copy(z_hbm, acc.at[pl.ds(sid * spt, spt)])
        plsc.subcore_barrier()

        def chunk(j, carry):
            pltpu.sync_copy(ones_v, acc.at[dst_s.at[j]], add=True)
            return carry

        lax.fori_loop(0, rpt, chunk, 0)
        plsc.subcore_barrier()
        pltpu.sync_copy(acc.at[pl.ds(sid * spt, spt)],
                        out_hbm.at[cid, pl.ds(sid * spt, spt)])

    ones = jnp.ones((_CH, _DW), jnp.float32)
    zrows = jnp.zeros((spt, _DW), jnp.float32)
    fn = pl.kernel(
        body,
        out_type=jax.ShapeDtypeStruct((_NC, nacc, _DW), jnp.float32),
        mesh=mesh,
        scratch_types=[
            pltpu.VMEM((rpt, _CH), jnp.int32),
            pltpu.VMEM((_CH, _DW), jnp.float32),
            pltpu.VMEM_SHARED((nacc, _DW), jnp.float32),
        ],
        interpret=interpret,
    )
    return fn(dst2d, ones, zrows)


def _sc_aggregate(src2d, dst2d, y0, y1, nacc, interpret=False):
    rows_total = src2d.shape[0]
    rpt = rows_total // _NS
    spt = nacc // _NS
    h2 = y0.shape[1]
    mesh = plsc.VectorSubcoreMesh(core_axis_name="c", subcore_axis_name="s")

    def body(src_hbm, dst_hbm, y0_hbm, y1_hbm, z_hbm, out_hbm,
             src_s, dst_s, rows_v, acc, sem):
        cid = lax.axis_index("c")
        sid = lax.axis_index("s")
        base = sid * rpt
        pltpu.sync_copy(src_hbm.at[pl.ds(base, rpt)], src_s)
        pltpu.sync_copy(dst_hbm.at[pl.ds(base, rpt)], dst_s)
        pltpu.sync_copy(z_hbm, acc.at[pl.ds(sid * spt, spt)])
        plsc.subcore_barrier()

        def run(tbl):
            def chunk(j, carry):
                pltpu.async_copy(tbl.at[src_s.at[j]], rows_v, sem).wait()
                pltpu.sync_copy(rows_v, acc.at[dst_s.at[j]], add=True)
                return carry

            lax.fori_loop(0, rpt, chunk, 0)

        @pl.when(cid == 0)
        def _():
            run(y0_hbm)

        @pl.when(cid == 1)
        def _():
            run(y1_hbm)

        plsc.subcore_barrier()
        pltpu.sync_copy(acc.at[pl.ds(sid * spt, spt)],
                        out_hbm.at[cid, pl.ds(sid * spt, spt)])

    zrows = jnp.zeros((spt, h2), jnp.float32)
    fn = pl.kernel(
        body,
        out_type=jax.ShapeDtypeStruct((_NC, nacc, h2), jnp.float32),
        mesh=mesh,
        scratch_types=[
            pltpu.VMEM((rpt, _CH), jnp.int32),
            pltpu.VMEM((rpt, _CH), jnp.int32),
            pltpu.VMEM((_CH, h2), jnp.float32),
            pltpu.VMEM_SHARED((nacc, h2), jnp.float32),
            pltpu.SemaphoreType.DMA,
        ],
        interpret=interpret,
    )
    return fn(src2d, dst2d, y0, y1, zrows)


def _dinv_from(deg_ref, blk):
    deg = deg_ref[0][:, 0] + deg_ref[1][:, 0] + 1.0
    return lax.rsqrt(deg)


def _tc_matmul_scaled(x, w, degp, nblk, blk, interpret=False):
    n, d = x.shape
    h2 = w.shape[1] // 2

    def body(x_ref, w_ref, deg_ref, y_ref):
        dinv = _dinv_from(deg_ref, blk)
        y = jnp.dot(x_ref[...], w_ref[...],
                    preferred_element_type=jnp.float32) * dinv[:, None]
        y_ref[0] = y[:, :h2]
        y_ref[1] = y[:, h2:]

    return pl.pallas_call(
        body,
        grid=(nblk,),
        in_specs=[
            pl.BlockSpec((blk, d), lambda i: (i, 0)),
            pl.BlockSpec((d, 2 * h2), lambda i: (0, 0)),
            pl.BlockSpec((2, blk, _DW), lambda i: (0, i, 0)),
        ],
        out_specs=pl.BlockSpec((2, blk, h2), lambda i: (0, i, 0)),
        out_shape=jax.ShapeDtypeStruct((2, n, h2), jnp.float32),
        interpret=interpret,
    )(x, w, degp)


def _tc_conv_stats(S, y, degp, bias2, nblk, blk, interpret=False):
    n = y.shape[1]
    h2 = y.shape[2]

    def body(s_ref, y_ref, deg_ref, b_ref, conv_ref, st_ref):
        i = pl.program_id(0)
        dinv = _dinv_from(deg_ref, blk)

        @pl.when(i == 0)
        def _():
            st_ref[...] = jnp.zeros_like(st_ref)

        for c in range(2):
            conv = (s_ref[c] + y_ref[c]) * dinv[:, None] + b_ref[c]
            conv_ref[c] = conv
            upd = jnp.concatenate(
                [jnp.sum(conv, axis=0)[None, :],
                 jnp.sum(conv * conv, axis=0)[None, :],
                 jnp.zeros((6, h2), jnp.float32)], axis=0)
            st_ref[c] = st_ref[c] + upd

    return pl.pallas_call(
        body,
        grid=(nblk,),
        in_specs=[
            pl.BlockSpec((2, blk, h2), lambda i: (0, i, 0)),
            pl.BlockSpec((2, blk, h2), lambda i: (0, i, 0)),
            pl.BlockSpec((2, blk, _DW), lambda i: (0, i, 0)),
            pl.BlockSpec((2, 1, h2), lambda i: (0, 0, 0)),
        ],
        out_specs=[
            pl.BlockSpec((2, blk, h2), lambda i: (0, i, 0)),
            pl.BlockSpec((2, 8, h2), lambda i: (0, 0, 0)),
        ],
        out_shape=[
            jax.ShapeDtypeStruct((2, n, h2), jnp.float32),
            jax.ShapeDtypeStruct((2, 8, h2), jnp.float32),
        ],
        interpret=interpret,
    )(S, y, degp, bias2)


def _bn_relu_halves(conv_ref, st_ref, g_ref, be_ref, n_nodes, h2):
    hs = []
    for c in range(2):
        mean = st_ref[c, 0:1, :] / n_nodes
        var = st_ref[c, 1:2, :] / n_nodes - mean * mean
        rstd = lax.rsqrt(var + _EPS)
        xn = (conv_ref[c] - mean) * rstd
        hs.append(jnp.maximum(xn * g_ref[c] + be_ref[c], 0.0))
    return jnp.concatenate(hs, axis=1)


def _tc_bn_matmul_scaled(conv, st, gam, bet, w, degp, n_nodes, nblk, blk,
                         interpret=False):
    n = conv.shape[1]
    h2 = conv.shape[2]

    def body(conv_ref, st_ref, g_ref, be_ref, w_ref, deg_ref, y_ref):
        dinv = _dinv_from(deg_ref, blk)
        hcat = _bn_relu_halves(conv_ref, st_ref, g_ref, be_ref, n_nodes, h2)
        y = jnp.dot(hcat, w_ref[...],
                    preferred_element_type=jnp.float32) * dinv[:, None]
        y_ref[0] = y[:, :h2]
        y_ref[1] = y[:, h2:]

    return pl.pallas_call(
        body,
        grid=(nblk,),
        in_specs=[
            pl.BlockSpec((2, blk, h2), lambda i: (0, i, 0)),
            pl.BlockSpec((2, 8, h2), lambda i: (0, 0, 0)),
            pl.BlockSpec((2, 1, h2), lambda i: (0, 0, 0)),
            pl.BlockSpec((2, 1, h2), lambda i: (0, 0, 0)),
            pl.BlockSpec((2 * h2, 2 * h2), lambda i: (0, 0)),
            pl.BlockSpec((2, blk, _DW), lambda i: (0, i, 0)),
        ],
        out_specs=pl.BlockSpec((2, blk, h2), lambda i: (0, i, 0)),
        out_shape=jax.ShapeDtypeStruct((2, n, h2), jnp.float32),
        interpret=interpret,
    )(conv, st, gam, bet, w, degp)


def _tc_bn_pool_head(conv, st, gam, bet, batch3, whp, bhp, n_nodes, n_graphs,
                     nblk, blk, interpret=False):
    h2 = conv.shape[2]
    hp = whp.shape[1]

    def body(conv_ref, st_ref, g_ref, be_ref, b3_ref, wh_ref, bh_ref,
             out_ref, psum, cnt):
        i = pl.program_id(0)
        hcat = _bn_relu_halves(conv_ref, st_ref, g_ref, be_ref, n_nodes, h2)
        bb = b3_ref[0, 0, :]
        oh = (bb[:, None] == lax.broadcasted_iota(
            jnp.int32, (blk, n_graphs), 1)).astype(jnp.float32)

        @pl.when(i == 0)
        def _():
            psum[...] = jnp.zeros_like(psum)
            cnt[...] = jnp.zeros_like(cnt)

        dn = (((0,), (0,)), ((), ()))
        psum[...] += lax.dot_general(oh, hcat, dn,
                                     preferred_element_type=jnp.float32)
        cnt[...] += lax.dot_general(oh, jnp.ones((blk, hp), jnp.float32), dn,
                                    preferred_element_type=jnp.float32)

        @pl.when(i == nblk - 1)
        def _():
            pooled = psum[...] / jnp.maximum(cnt[...][:, 0:1], 1.0)
            out_ref[...] = jnp.dot(pooled, wh_ref[...],
                                   preferred_element_type=jnp.float32) + bh_ref[...]

    return pl.pallas_call(
        body,
        grid=(nblk,),
        in_specs=[
            pl.BlockSpec((2, blk, h2), lambda i: (0, i, 0)),
            pl.BlockSpec((2, 8, h2), lambda i: (0, 0, 0)),
            pl.BlockSpec((2, 1, h2), lambda i: (0, 0, 0)),
            pl.BlockSpec((2, 1, h2), lambda i: (0, 0, 0)),
            pl.BlockSpec((1, 1, blk), lambda i: (i, 0, 0)),
            pl.BlockSpec((2 * h2, hp), lambda i: (0, 0)),
            pl.BlockSpec((1, hp), lambda i: (0, 0)),
        ],
        out_specs=pl.BlockSpec((n_graphs, hp), lambda i: (0, 0)),
        out_shape=jax.ShapeDtypeStruct((n_graphs, hp), jnp.float32),
        scratch_shapes=[
            pltpu.VMEM((n_graphs, 2 * h2), jnp.float32),
            pltpu.VMEM((n_graphs, hp), jnp.float32),
        ],
        interpret=interpret,
    )(conv, st, gam, bet, batch3, whp, bhp)


def kernel(graph_x, edge_index, batch, W1, b1, gamma1, beta1,
           W2, b2, gamma2, beta2, Wh, bh):
    n, d = graph_x.shape
    e = edge_index.shape[1]
    h = W1.shape[1]
    h2 = h // 2
    o = Wh.shape[1]
    n_graphs = 64
    blk = 400
    nblk = n // blk

    rows = _round_up(e, _CH * _NC * _NS) // _CH
    pad = rows * _CH - e
    nacc = _round_up(n + 1, 128)

    src = edge_index[0]
    dst = edge_index[1]
    src2d = jnp.concatenate(
        [src, jnp.zeros((pad,), src.dtype)]).reshape(rows, _CH)
    dst2d = jnp.concatenate(
        [dst, jnp.full((pad,), n, dst.dtype)]).reshape(rows, _CH)

    b1h = b1.reshape(2, 1, h2)
    g1h = gamma1.reshape(2, 1, h2)
    be1h = beta1.reshape(2, 1, h2)
    b2h = b2.reshape(2, 1, h2)
    g2h = gamma2.reshape(2, 1, h2)
    be2h = beta2.reshape(2, 1, h2)

    degp = _sc_degree(dst2d, nacc)

    y1 = _tc_matmul_scaled(graph_x.astype(jnp.float32), W1, degp, nblk, blk)
    S1 = _sc_aggregate(src2d, dst2d, y1[0], y1[1], nacc)
    c1, st1 = _tc_conv_stats(S1, y1, degp, b1h, nblk, blk)

    y2 = _tc_bn_matmul_scaled(c1, st1, g1h, be1h, W2, degp, float(n),
                              nblk, blk)
    S2 = _sc_aggregate(src2d, dst2d, y2[0], y2[1], nacc)
    c2, st2 = _tc_conv_stats(S2, y2, degp, b2h, nblk, blk)

    batch3 = batch.reshape(nblk, 1, blk)
    hp = 128
    whp = jnp.zeros((h, hp), jnp.float32).at[:, :o].set(Wh)
    bhp = jnp.zeros((1, hp), jnp.float32).at[0, :o].set(bh)
    outp = _tc_bn_pool_head(c2, st2, g2h, be2h, batch3, whp, bhp,
                            float(n), n_graphs, nblk, blk)
    return outp[:, :o]


# speedup vs baseline: 7.2604x; 1.1136x over previous
import jax
import jax.numpy as jnp
from jax import lax
from jax.experimental import pallas as pl
from jax.experimental.pallas import tpu as pltpu
from jax.experimental.pallas import tpu_sc as plsc

_NC = 2
_NS = 16
_CH = 128
_EPS = 1e-5


def _round_up(x, m):
    return (x + m - 1) // m * m


_DW = 128


def _sc_degree(dst2d, nacc, interpret=False):
    rows_total = dst2d.shape[0]
    rpt = rows_total // (_NC * _NS)
    spt = nacc // _NS
    mesh = plsc.VectorSubcoreMesh(core_axis_name="c", subcore_axis_name="s")

    def body(dst_hbm, ones_hbm, z_hbm, out_hbm, dst_s, ones_v, acc):
        cid = lax.axis_index("c")
        sid = lax.axis_index("s")
        base = (cid * _NS + sid) * rpt
        pltpu.sync_copy(dst_hbm.at[pl.ds(base, rpt)], dst_s)
        pltpu.sync_copy(ones_hbm, ones_v)
        pltpu.sync_copy(z_hbm, acc.at[pl.ds(sid * spt, spt)])
        plsc.subcore_barrier()

        def chunk(j, carry):
            pltpu.sync_copy(ones_v, acc.at[dst_s.at[j]], add=True)
            return carry

        lax.fori_loop(0, rpt, chunk, 0)
        plsc.subcore_barrier()
        pltpu.sync_copy(acc.at[pl.ds(sid * spt, spt)],
                        out_hbm.at[cid, pl.ds(sid * spt, spt)])

    ones = jnp.ones((_CH, _DW), jnp.float32)
    zrows = jnp.zeros((spt, _DW), jnp.float32)
    fn = pl.kernel(
        body,
        out_type=jax.ShapeDtypeStruct((_NC, nacc, _DW), jnp.float32),
        mesh=mesh,
        scratch_types=[
            pltpu.VMEM((rpt, _CH), jnp.int32),
            pltpu.VMEM((_CH, _DW), jnp.float32),
            pltpu.VMEM_SHARED((nacc, _DW), jnp.float32),
        ],
        interpret=interpret,
    )
    return fn(dst2d, ones, zrows)


def _sc_aggregate(src2d, dst2d, y0, y1, nacc, interpret=False):
    rows_total = src2d.shape[0]
    rpt = rows_total // _NS
    spt = nacc // _NS
    h2 = y0.shape[1]
    mesh = plsc.VectorSubcoreMesh(core_axis_name="c", subcore_axis_name="s")

    def body(src_hbm, dst_hbm, y0_hbm, y1_hbm, z_hbm, out_hbm,
             src_s, dst_r, rows_v, acc, sem, dsem):
        cid = lax.axis_index("c")
        sid = lax.axis_index("s")
        base = sid * rpt
        pltpu.sync_copy(src_hbm.at[pl.ds(base, rpt)], src_s)
        pltpu.sync_copy(z_hbm, acc.at[pl.ds(sid * spt, spt)])
        plsc.subcore_barrier()

        def run(tbl):
            pltpu.async_copy(dst_hbm.at[base], dst_r.at[0], dsem)
            pltpu.async_copy(tbl.at[src_s.at[0]], rows_v.at[0], sem)

            def chunk(j, carry):
                slot = lax.rem(j, 2)
                pltpu.make_async_copy(dst_hbm.at[base + j],
                                      dst_r.at[slot], dsem).wait()
                pltpu.make_async_copy(tbl.at[src_s.at[j]],
                                      rows_v.at[slot], sem).wait()

                @pl.when(j + 1 < rpt)
                def _():
                    pltpu.async_copy(dst_hbm.at[base + j + 1],
                                     dst_r.at[1 - slot], dsem)
                    pltpu.async_copy(tbl.at[src_s.at[j + 1]],
                                     rows_v.at[1 - slot], sem)

                pltpu.sync_copy(rows_v.at[slot], acc.at[dst_r.at[slot]],
                                add=True)
                return carry

            lax.fori_loop(0, rpt, chunk, 0)

        @pl.when(cid == 0)
        def _():
            run(y0_hbm)

        @pl.when(cid == 1)
        def _():
            run(y1_hbm)

        plsc.subcore_barrier()
        pltpu.sync_copy(acc.at[pl.ds(sid * spt, spt)],
                        out_hbm.at[cid, pl.ds(sid * spt, spt)])

    zrows = jnp.zeros((spt, h2), jnp.float32)
    fn = pl.kernel(
        body,
        out_type=jax.ShapeDtypeStruct((_NC, nacc, h2), jnp.float32),
        mesh=mesh,
        scratch_types=[
            pltpu.VMEM((rpt, _CH), jnp.int32),
            pltpu.VMEM((2, _CH), jnp.int32),
            pltpu.VMEM((2, _CH, h2), jnp.float32),
            pltpu.VMEM_SHARED((nacc, h2), jnp.float32),
            pltpu.SemaphoreType.DMA,
            pltpu.SemaphoreType.DMA,
        ],
        interpret=interpret,
    )
    return fn(src2d, dst2d, y0, y1, zrows)


def _dinv_from(deg_ref, blk):
    deg = deg_ref[0][:, 0] + deg_ref[1][:, 0] + 1.0
    return lax.rsqrt(deg)


def _tc_matmul_scaled(x, w, degp, nblk, blk, interpret=False):
    n, d = x.shape
    h2 = w.shape[1] // 2

    def body(x_ref, w_ref, deg_ref, y_ref):
        dinv = _dinv_from(deg_ref, blk)
        y = jnp.dot(x_ref[...], w_ref[...],
                    preferred_element_type=jnp.float32) * dinv[:, None]
        y_ref[0] = y[:, :h2]
        y_ref[1] = y[:, h2:]

    return pl.pallas_call(
        body,
        grid=(nblk,),
        in_specs=[
            pl.BlockSpec((blk, d), lambda i: (i, 0)),
            pl.BlockSpec((d, 2 * h2), lambda i: (0, 0)),
            pl.BlockSpec((2, blk, _DW), lambda i: (0, i, 0)),
        ],
        out_specs=pl.BlockSpec((2, blk, h2), lambda i: (0, i, 0)),
        out_shape=jax.ShapeDtypeStruct((2, n, h2), jnp.float32),
        interpret=interpret,
    )(x, w, degp)


def _tc_conv_stats(S, y, degp, bias2, nblk, blk, interpret=False):
    n = y.shape[1]
    h2 = y.shape[2]

    def body(s_ref, y_ref, deg_ref, b_ref, conv_ref, st_ref):
        i = pl.program_id(0)
        dinv = _dinv_from(deg_ref, blk)

        @pl.when(i == 0)
        def _():
            st_ref[...] = jnp.zeros_like(st_ref)

        for c in range(2):
            conv = (s_ref[c] + y_ref[c]) * dinv[:, None] + b_ref[c]
            conv_ref[c] = conv
            upd = jnp.concatenate(
                [jnp.sum(conv, axis=0)[None, :],
                 jnp.sum(conv * conv, axis=0)[None, :],
                 jnp.zeros((6, h2), jnp.float32)], axis=0)
            st_ref[c] = st_ref[c] + upd

    return pl.pallas_call(
        body,
        grid=(nblk,),
        in_specs=[
            pl.BlockSpec((2, blk, h2), lambda i: (0, i, 0)),
            pl.BlockSpec((2, blk, h2), lambda i: (0, i, 0)),
            pl.BlockSpec((2, blk, _DW), lambda i: (0, i, 0)),
            pl.BlockSpec((2, 1, h2), lambda i: (0, 0, 0)),
        ],
        out_specs=[
            pl.BlockSpec((2, blk, h2), lambda i: (0, i, 0)),
            pl.BlockSpec((2, 8, h2), lambda i: (0, 0, 0)),
        ],
        out_shape=[
            jax.ShapeDtypeStruct((2, n, h2), jnp.float32),
            jax.ShapeDtypeStruct((2, 8, h2), jnp.float32),
        ],
        interpret=interpret,
    )(S, y, degp, bias2)


def _bn_relu_halves(conv_ref, st_ref, g_ref, be_ref, n_nodes, h2):
    hs = []
    for c in range(2):
        mean = st_ref[c, 0:1, :] / n_nodes
        var = st_ref[c, 1:2, :] / n_nodes - mean * mean
        rstd = lax.rsqrt(var + _EPS)
        xn = (conv_ref[c] - mean) * rstd
        hs.append(jnp.maximum(xn * g_ref[c] + be_ref[c], 0.0))
    return jnp.concatenate(hs, axis=1)


def _tc_bn_matmul_scaled(conv, st, gam, bet, w, degp, n_nodes, nblk, blk,
                         interpret=False):
    n = conv.shape[1]
    h2 = conv.shape[2]

    def body(conv_ref, st_ref, g_ref, be_ref, w_ref, deg_ref, y_ref):
        dinv = _dinv_from(deg_ref, blk)
        hcat = _bn_relu_halves(conv_ref, st_ref, g_ref, be_ref, n_nodes, h2)
        y = jnp.dot(hcat, w_ref[...],
                    preferred_element_type=jnp.float32) * dinv[:, None]
        y_ref[0] = y[:, :h2]
        y_ref[1] = y[:, h2:]

    return pl.pallas_call(
        body,
        grid=(nblk,),
        in_specs=[
            pl.BlockSpec((2, blk, h2), lambda i: (0, i, 0)),
            pl.BlockSpec((2, 8, h2), lambda i: (0, 0, 0)),
            pl.BlockSpec((2, 1, h2), lambda i: (0, 0, 0)),
            pl.BlockSpec((2, 1, h2), lambda i: (0, 0, 0)),
            pl.BlockSpec((2 * h2, 2 * h2), lambda i: (0, 0)),
            pl.BlockSpec((2, blk, _DW), lambda i: (0, i, 0)),
        ],
        out_specs=pl.BlockSpec((2, blk, h2), lambda i: (0, i, 0)),
        out_shape=jax.ShapeDtypeStruct((2, n, h2), jnp.float32),
        interpret=interpret,
    )(conv, st, gam, bet, w, degp)


def _tc_bn_pool_head(conv, st, gam, bet, batch3, whp, bhp, n_nodes, n_graphs,
                     nblk, blk, interpret=False):
    h2 = conv.shape[2]
    hp = whp.shape[1]

    def body(conv_ref, st_ref, g_ref, be_ref, b3_ref, wh_ref, bh_ref,
             out_ref, psum, cnt):
        i = pl.program_id(0)
        hcat = _bn_relu_halves(conv_ref, st_ref, g_ref, be_ref, n_nodes, h2)
        bb = b3_ref[0, 0, :]
        oh = (bb[:, None] == lax.broadcasted_iota(
            jnp.int32, (blk, n_graphs), 1)).astype(jnp.float32)

        @pl.when(i == 0)
        def _():
            psum[...] = jnp.zeros_like(psum)
            cnt[...] = jnp.zeros_like(cnt)

        dn = (((0,), (0,)), ((), ()))
        psum[...] += lax.dot_general(oh, hcat, dn,
                                     preferred_element_type=jnp.float32)
        cnt[...] += lax.dot_general(oh, jnp.ones((blk, hp), jnp.float32), dn,
                                    preferred_element_type=jnp.float32)

        @pl.when(i == nblk - 1)
        def _():
            pooled = psum[...] / jnp.maximum(cnt[...][:, 0:1], 1.0)
            out_ref[...] = jnp.dot(pooled, wh_ref[...],
                                   preferred_element_type=jnp.float32) + bh_ref[...]

    return pl.pallas_call(
        body,
        grid=(nblk,),
        in_specs=[
            pl.BlockSpec((2, blk, h2), lambda i: (0, i, 0)),
            pl.BlockSpec((2, 8, h2), lambda i: (0, 0, 0)),
            pl.BlockSpec((2, 1, h2), lambda i: (0, 0, 0)),
            pl.BlockSpec((2, 1, h2), lambda i: (0, 0, 0)),
            pl.BlockSpec((1, 1, blk), lambda i: (i, 0, 0)),
            pl.BlockSpec((2 * h2, hp), lambda i: (0, 0)),
            pl.BlockSpec((1, hp), lambda i: (0, 0)),
        ],
        out_specs=pl.BlockSpec((n_graphs, hp), lambda i: (0, 0)),
        out_shape=jax.ShapeDtypeStruct((n_graphs, hp), jnp.float32),
        scratch_shapes=[
            pltpu.VMEM((n_graphs, 2 * h2), jnp.float32),
            pltpu.VMEM((n_graphs, hp), jnp.float32),
        ],
        interpret=interpret,
    )(conv, st, gam, bet, batch3, whp, bhp)


def kernel(graph_x, edge_index, batch, W1, b1, gamma1, beta1,
           W2, b2, gamma2, beta2, Wh, bh):
    n, d = graph_x.shape
    e = edge_index.shape[1]
    h = W1.shape[1]
    h2 = h // 2
    o = Wh.shape[1]
    n_graphs = 64
    blk = 400
    nblk = n // blk

    rows = _round_up(e, _CH * _NC * _NS) // _CH
    pad = rows * _CH - e
    nacc = _round_up(n + 1, 128)

    src = edge_index[0]
    dst = edge_index[1]
    src2d = jnp.concatenate(
        [src, jnp.zeros((pad,), src.dtype)]).reshape(rows, _CH)
    dst2d = jnp.concatenate(
        [dst, jnp.full((pad,), n, dst.dtype)]).reshape(rows, _CH)

    b1h = b1.reshape(2, 1, h2)
    g1h = gamma1.reshape(2, 1, h2)
    be1h = beta1.reshape(2, 1, h2)
    b2h = b2.reshape(2, 1, h2)
    g2h = gamma2.reshape(2, 1, h2)
    be2h = beta2.reshape(2, 1, h2)

    degp = _sc_degree(dst2d, nacc)

    y1 = _tc_matmul_scaled(graph_x.astype(jnp.float32), W1, degp, nblk, blk)
    S1 = _sc_aggregate(src2d, dst2d, y1[0], y1[1], nacc)
    c1, st1 = _tc_conv_stats(S1, y1, degp, b1h, nblk, blk)

    y2 = _tc_bn_matmul_scaled(c1, st1, g1h, be1h, W2, degp, float(n),
                              nblk, blk)
    S2 = _sc_aggregate(src2d, dst2d, y2[0], y2[1], nacc)
    c2, st2 = _tc_conv_stats(S2, y2, degp, b2h, nblk, blk)

    batch3 = batch.reshape(nblk, 1, blk)
    hp = 128
    whp = jnp.zeros((h, hp), jnp.float32).at[:, :o].set(Wh)
    bhp = jnp.zeros((1, hp), jnp.float32).at[0, :o].set(bh)
    outp = _tc_bn_pool_head(c2, st2, g2h, be2h, batch3, whp, bhp,
                            float(n), n_graphs, nblk, blk)
    return outp[:, :o]


# speedup vs baseline: 7.6435x; 1.0528x over previous
import jax
import jax.numpy as jnp
from jax import lax
from jax.experimental import pallas as pl
from jax.experimental.pallas import tpu as pltpu
from jax.experimental.pallas import tpu_sc as plsc

_NC = 2
_NS = 16
_CH = 128
_EPS = 1e-5


def _round_up(x, m):
    return (x + m - 1) // m * m


_DW = 128


def _sc_degree(dst2d, nacc, interpret=False):
    rows_total = dst2d.shape[0]
    rpt = rows_total // (_NC * _NS)
    spt = nacc // _NS
    mesh = plsc.VectorSubcoreMesh(core_axis_name="c", subcore_axis_name="s")

    def body(dst_hbm, ones_hbm, z_hbm, out_hbm, dst_s, ones_v, acc):
        cid = lax.axis_index("c")
        sid = lax.axis_index("s")
        base = (cid * _NS + sid) * rpt
        pltpu.sync_copy(dst_hbm.at[pl.ds(base, rpt)], dst_s)
        pltpu.sync_copy(ones_hbm, ones_v)
        pltpu.sync_copy(z_hbm, acc.at[pl.ds(sid * spt, spt)])
        plsc.subcore_barrier()

        def chunk(j, carry):
            pltpu.sync_copy(ones_v, acc.at[dst_s.at[j]], add=True)
            return carry

        lax.fori_loop(0, rpt, chunk, 0)
        plsc.subcore_barrier()
        pltpu.sync_copy(acc.at[pl.ds(sid * spt, spt)],
                        out_hbm.at[cid, pl.ds(sid * spt, spt)])

    ones = jnp.ones((_CH, _DW), jnp.float32)
    zrows = jnp.zeros((spt, _DW), jnp.float32)
    fn = pl.kernel(
        body,
        out_type=jax.ShapeDtypeStruct((_NC, nacc, _DW), jnp.float32),
        mesh=mesh,
        scratch_types=[
            pltpu.VMEM((rpt, _CH), jnp.int32),
            pltpu.VMEM((_CH, _DW), jnp.float32),
            pltpu.VMEM_SHARED((nacc, _DW), jnp.float32),
        ],
        interpret=interpret,
    )
    return fn(dst2d, ones, zrows)


def _sc_aggregate(src2d, dst2d, y0, y1, nacc, interpret=False):
    rows_total = src2d.shape[0]
    rpt = rows_total // _NS
    spt = nacc // _NS
    h2 = y0.shape[1]
    mesh = plsc.VectorSubcoreMesh(core_axis_name="c", subcore_axis_name="s")

    def body(src_hbm, dst_hbm, y0_hbm, y1_hbm, z_hbm, out_hbm,
             src_r, dst_r, rows_v, acc, isem, dsem, sem, ssem):
        cid = lax.axis_index("c")
        sid = lax.axis_index("s")
        base = sid * rpt
        pltpu.sync_copy(z_hbm, acc.at[pl.ds(sid * spt, spt)])
        plsc.subcore_barrier()

        def run(tbl):
            for k in range(2):
                pltpu.async_copy(src_hbm.at[base + k], src_r.at[k], isem)
                pltpu.async_copy(dst_hbm.at[base + k], dst_r.at[k], dsem)
            pltpu.make_async_copy(src_hbm.at[base], src_r.at[0], isem).wait()
            pltpu.async_copy(tbl.at[src_r.at[0]], rows_v.at[0], sem)

            def chunk(j, carry):
                r = lax.rem(j, 3)

                @pl.when(j >= 1)
                def _():
                    pltpu.make_async_copy(rows_v.at[0],
                                          acc.at[dst_r.at[0]], ssem).wait()

                @pl.when(j + 2 < rpt)
                def _():
                    s3 = lax.rem(j + 2, 3)
                    pltpu.async_copy(src_hbm.at[base + j + 2],
                                     src_r.at[s3], isem)
                    pltpu.async_copy(dst_hbm.at[base + j + 2],
                                     dst_r.at[s3], dsem)

                @pl.when(j + 1 < rpt)
                def _():
                    pltpu.make_async_copy(src_hbm.at[base + j + 1],
                                          src_r.at[0], isem).wait()
                    nr = lax.rem(j + 1, 3)
                    pltpu.async_copy(tbl.at[src_r.at[nr]],
                                     rows_v.at[nr], sem)

                pltpu.make_async_copy(tbl.at[src_r.at[0]],
                                      rows_v.at[r], sem).wait()
                pltpu.make_async_copy(dst_hbm.at[base + j],
                                      dst_r.at[0], dsem).wait()
                pltpu.async_copy(rows_v.at[r], acc.at[dst_r.at[r]],
                                 ssem, add=True)
                return carry

            lax.fori_loop(0, rpt, chunk, 0)
            pltpu.make_async_copy(rows_v.at[0], acc.at[dst_r.at[0]],
                                  ssem).wait()

        @pl.when(cid == 0)
        def _():
            run(y0_hbm)

        @pl.when(cid == 1)
        def _():
            run(y1_hbm)

        plsc.subcore_barrier()
        pltpu.sync_copy(acc.at[pl.ds(sid * spt, spt)],
                        out_hbm.at[cid, pl.ds(sid * spt, spt)])

    zrows = jnp.zeros((spt, h2), jnp.float32)
    fn = pl.kernel(
        body,
        out_type=jax.ShapeDtypeStruct((_NC, nacc, h2), jnp.float32),
        mesh=mesh,
        scratch_types=[
            pltpu.VMEM((3, _CH), jnp.int32),
            pltpu.VMEM((3, _CH), jnp.int32),
            pltpu.VMEM((3, _CH, h2), jnp.float32),
            pltpu.VMEM_SHARED((nacc, h2), jnp.float32),
            pltpu.SemaphoreType.DMA,
            pltpu.SemaphoreType.DMA,
            pltpu.SemaphoreType.DMA,
            pltpu.SemaphoreType.DMA,
        ],
        interpret=interpret,
    )
    return fn(src2d, dst2d, y0, y1, zrows)


def _dinv_from(deg_ref, blk):
    deg = deg_ref[0][:, 0] + deg_ref[1][:, 0] + 1.0
    return lax.rsqrt(deg)


def _tc_matmul_scaled(x, w, degp, nblk, blk, interpret=False):
    n, d = x.shape
    h2 = w.shape[1] // 2

    def body(x_ref, w_ref, deg_ref, y_ref):
        dinv = _dinv_from(deg_ref, blk)
        y = jnp.dot(x_ref[...], w_ref[...],
                    preferred_element_type=jnp.float32) * dinv[:, None]
        y_ref[0] = y[:, :h2]
        y_ref[1] = y[:, h2:]

    return pl.pallas_call(
        body,
        grid=(nblk,),
        in_specs=[
            pl.BlockSpec((blk, d), lambda i: (i, 0)),
            pl.BlockSpec((d, 2 * h2), lambda i: (0, 0)),
            pl.BlockSpec((2, blk, _DW), lambda i: (0, i, 0)),
        ],
        out_specs=pl.BlockSpec((2, blk, h2), lambda i: (0, i, 0)),
        out_shape=jax.ShapeDtypeStruct((2, n, h2), jnp.float32),
        interpret=interpret,
    )(x, w, degp)


def _tc_conv_stats(S, y, degp, bias2, nblk, blk, interpret=False):
    n = y.shape[1]
    h2 = y.shape[2]

    def body(s_ref, y_ref, deg_ref, b_ref, conv_ref, st_ref):
        i = pl.program_id(0)
        dinv = _dinv_from(deg_ref, blk)

        @pl.when(i == 0)
        def _():
            st_ref[...] = jnp.zeros_like(st_ref)

        for c in range(2):
            conv = (s_ref[c] + y_ref[c]) * dinv[:, None] + b_ref[c]
            conv_ref[c] = conv
            upd = jnp.concatenate(
                [jnp.sum(conv, axis=0)[None, :],
                 jnp.sum(conv * conv, axis=0)[None, :],
                 jnp.zeros((6, h2), jnp.float32)], axis=0)
            st_ref[c] = st_ref[c] + upd

    return pl.pallas_call(
        body,
        grid=(nblk,),
        in_specs=[
            pl.BlockSpec((2, blk, h2), lambda i: (0, i, 0)),
            pl.BlockSpec((2, blk, h2), lambda i: (0, i, 0)),
            pl.BlockSpec((2, blk, _DW), lambda i: (0, i, 0)),
            pl.BlockSpec((2, 1, h2), lambda i: (0, 0, 0)),
        ],
        out_specs=[
            pl.BlockSpec((2, blk, h2), lambda i: (0, i, 0)),
            pl.BlockSpec((2, 8, h2), lambda i: (0, 0, 0)),
        ],
        out_shape=[
            jax.ShapeDtypeStruct((2, n, h2), jnp.float32),
            jax.ShapeDtypeStruct((2, 8, h2), jnp.float32),
        ],
        interpret=interpret,
    )(S, y, degp, bias2)


def _bn_relu_halves(conv_ref, st_ref, g_ref, be_ref, n_nodes, h2):
    hs = []
    for c in range(2):
        mean = st_ref[c, 0:1, :] / n_nodes
        var = st_ref[c, 1:2, :] / n_nodes - mean * mean
        rstd = lax.rsqrt(var + _EPS)
        xn = (conv_ref[c] - mean) * rstd
        hs.append(jnp.maximum(xn * g_ref[c] + be_ref[c], 0.0))
    return jnp.concatenate(hs, axis=1)


def _tc_bn_matmul_scaled(conv, st, gam, bet, w, degp, n_nodes, nblk, blk,
                         interpret=False):
    n = conv.shape[1]
    h2 = conv.shape[2]

    def body(conv_ref, st_ref, g_ref, be_ref, w_ref, deg_ref, y_ref):
        dinv = _dinv_from(deg_ref, blk)
        hcat = _bn_relu_halves(conv_ref, st_ref, g_ref, be_ref, n_nodes, h2)
        y = jnp.dot(hcat, w_ref[...],
                    preferred_element_type=jnp.float32) * dinv[:, None]
        y_ref[0] = y[:, :h2]
        y_ref[1] = y[:, h2:]

    return pl.pallas_call(
        body,
        grid=(nblk,),
        in_specs=[
            pl.BlockSpec((2, blk, h2), lambda i: (0, i, 0)),
            pl.BlockSpec((2, 8, h2), lambda i: (0, 0, 0)),
            pl.BlockSpec((2, 1, h2), lambda i: (0, 0, 0)),
            pl.BlockSpec((2, 1, h2), lambda i: (0, 0, 0)),
            pl.BlockSpec((2 * h2, 2 * h2), lambda i: (0, 0)),
            pl.BlockSpec((2, blk, _DW), lambda i: (0, i, 0)),
        ],
        out_specs=pl.BlockSpec((2, blk, h2), lambda i: (0, i, 0)),
        out_shape=jax.ShapeDtypeStruct((2, n, h2), jnp.float32),
        interpret=interpret,
    )(conv, st, gam, bet, w, degp)


def _tc_bn_pool_head(conv, st, gam, bet, batch3, whp, bhp, n_nodes, n_graphs,
                     nblk, blk, interpret=False):
    h2 = conv.shape[2]
    hp = whp.shape[1]

    def body(conv_ref, st_ref, g_ref, be_ref, b3_ref, wh_ref, bh_ref,
             out_ref, psum, cnt):
        i = pl.program_id(0)
        hcat = _bn_relu_halves(conv_ref, st_ref, g_ref, be_ref, n_nodes, h2)
        bb = b3_ref[0, 0, :]
        oh = (bb[:, None] == lax.broadcasted_iota(
            jnp.int32, (blk, n_graphs), 1)).astype(jnp.float32)

        @pl.when(i == 0)
        def _():
            psum[...] = jnp.zeros_like(psum)
            cnt[...] = jnp.zeros_like(cnt)

        dn = (((0,), (0,)), ((), ()))
        psum[...] += lax.dot_general(oh, hcat, dn,
                                     preferred_element_type=jnp.float32)
        cnt[...] += lax.dot_general(oh, jnp.ones((blk, hp), jnp.float32), dn,
                                    preferred_element_type=jnp.float32)

        @pl.when(i == nblk - 1)
        def _():
            pooled = psum[...] / jnp.maximum(cnt[...][:, 0:1], 1.0)
            out_ref[...] = jnp.dot(pooled, wh_ref[...],
                                   preferred_element_type=jnp.float32) + bh_ref[...]

    return pl.pallas_call(
        body,
        grid=(nblk,),
        in_specs=[
            pl.BlockSpec((2, blk, h2), lambda i: (0, i, 0)),
            pl.BlockSpec((2, 8, h2), lambda i: (0, 0, 0)),
            pl.BlockSpec((2, 1, h2), lambda i: (0, 0, 0)),
            pl.BlockSpec((2, 1, h2), lambda i: (0, 0, 0)),
            pl.BlockSpec((1, 1, blk), lambda i: (i, 0, 0)),
            pl.BlockSpec((2 * h2, hp), lambda i: (0, 0)),
            pl.BlockSpec((1, hp), lambda i: (0, 0)),
        ],
        out_specs=pl.BlockSpec((n_graphs, hp), lambda i: (0, 0)),
        out_shape=jax.ShapeDtypeStruct((n_graphs, hp), jnp.float32),
        scratch_shapes=[
            pltpu.VMEM((n_graphs, 2 * h2), jnp.float32),
            pltpu.VMEM((n_graphs, hp), jnp.float32),
        ],
        interpret=interpret,
    )(conv, st, gam, bet, batch3, whp, bhp)


def kernel(graph_x, edge_index, batch, W1, b1, gamma1, beta1,
           W2, b2, gamma2, beta2, Wh, bh):
    n, d = graph_x.shape
    e = edge_index.shape[1]
    h = W1.shape[1]
    h2 = h // 2
    o = Wh.shape[1]
    n_graphs = 64
    blk = 400
    nblk = n // blk

    rows = _round_up(e, _CH * _NC * _NS) // _CH
    pad = rows * _CH - e
    nacc = _round_up(n + 1, 128)

    src = edge_index[0]
    dst = edge_index[1]
    src2d = jnp.concatenate(
        [src, jnp.zeros((pad,), src.dtype)]).reshape(rows, _CH)
    dst2d = jnp.concatenate(
        [dst, jnp.full((pad,), n, dst.dtype)]).reshape(rows, _CH)

    b1h = b1.reshape(2, 1, h2)
    g1h = gamma1.reshape(2, 1, h2)
    be1h = beta1.reshape(2, 1, h2)
    b2h = b2.reshape(2, 1, h2)
    g2h = gamma2.reshape(2, 1, h2)
    be2h = beta2.reshape(2, 1, h2)

    degp = _sc_degree(dst2d, nacc)

    y1 = _tc_matmul_scaled(graph_x.astype(jnp.float32), W1, degp, nblk, blk)
    S1 = _sc_aggregate(src2d, dst2d, y1[0], y1[1], nacc)
    c1, st1 = _tc_conv_stats(S1, y1, degp, b1h, nblk, blk)

    y2 = _tc_bn_matmul_scaled(c1, st1, g1h, be1h, W2, degp, float(n),
                              nblk, blk)
    S2 = _sc_aggregate(src2d, dst2d, y2[0], y2[1], nacc)
    c2, st2 = _tc_conv_stats(S2, y2, degp, b2h, nblk, blk)

    batch3 = batch.reshape(nblk, 1, blk)
    hp = 128
    whp = jnp.zeros((h, hp), jnp.float32).at[:, :o].set(Wh)
    bhp = jnp.zeros((1, hp), jnp.float32).at[0, :o].set(bh)
    outp = _tc_bn_pool_head(c2, st2, g2h, be2h, batch3, whp, bhp,
                            float(n), n_graphs, nblk, blk)
    return outp[:, :o]


# speedup vs baseline: 8.3910x; 1.0978x over previous
import jax
import jax.numpy as jnp
from jax import lax
from jax.experimental import pallas as pl
from jax.experimental.pallas import tpu as pltpu
from jax.experimental.pallas import tpu_sc as plsc

_NC = 2
_NS = 16
_CH = 128
_EPS = 1e-5


def _round_up(x, m):
    return (x + m - 1) // m * m


_DW = 128


def _sc_degree(dst2d, nacc, interpret=False):
    rows_total = dst2d.shape[0]
    rpt = rows_total // (_NC * _NS)
    spt = nacc // _NS
    mesh = plsc.VectorSubcoreMesh(core_axis_name="c", subcore_axis_name="s")

    def body(dst_hbm, ones_hbm, z_hbm, out_hbm, dst_s, ones_v, acc):
        cid = lax.axis_index("c")
        sid = lax.axis_index("s")
        base = (cid * _NS + sid) * rpt
        pltpu.sync_copy(dst_hbm.at[pl.ds(base, rpt)], dst_s)
        pltpu.sync_copy(ones_hbm, ones_v)
        pltpu.sync_copy(z_hbm, acc.at[pl.ds(sid * spt, spt)])
        plsc.subcore_barrier()

        def chunk(j, carry):
            pltpu.sync_copy(ones_v, acc.at[dst_s.at[j]], add=True)
            return carry

        lax.fori_loop(0, rpt, chunk, 0)
        plsc.subcore_barrier()
        pltpu.sync_copy(acc.at[pl.ds(sid * spt, spt)],
                        out_hbm.at[cid, pl.ds(sid * spt, spt)])

    ones = jnp.ones((_CH, _DW), jnp.float32)
    zrows = jnp.zeros((spt, _DW), jnp.float32)
    fn = pl.kernel(
        body,
        out_type=jax.ShapeDtypeStruct((_NC, nacc, _DW), jnp.float32),
        mesh=mesh,
        scratch_types=[
            pltpu.VMEM((rpt, _CH), jnp.int32),
            pltpu.VMEM((_CH, _DW), jnp.float32),
            pltpu.VMEM_SHARED((nacc, _DW), jnp.float32),
        ],
        interpret=interpret,
    )
    return fn(dst2d, ones, zrows)


def _sc_aggregate(src2d, dst2d, y0, y1, nacc, interpret=False):
    rows_total = src2d.shape[0]
    rpt = rows_total // _NS
    spt = nacc // _NS
    h2 = y0.shape[1]
    mesh = plsc.VectorSubcoreMesh(core_axis_name="c", subcore_axis_name="s")

    def body(src_hbm, dst_hbm, y0_hbm, y1_hbm, z_hbm, out_hbm,
             src_r, dst_r, rows_v, acc, isem, dsem, sem, ssem):
        cid = lax.axis_index("c")
        sid = lax.axis_index("s")
        base = sid * rpt
        pltpu.sync_copy(z_hbm, acc.at[pl.ds(sid * spt, spt)])
        plsc.subcore_barrier()

        def run(tbl):
            for k in range(2):
                pltpu.async_copy(src_hbm.at[base + k], src_r.at[k], isem)
                pltpu.async_copy(dst_hbm.at[base + k], dst_r.at[k], dsem)
            pltpu.make_async_copy(src_hbm.at[base], src_r.at[0], isem).wait()
            pltpu.async_copy(tbl.at[src_r.at[0]], rows_v.at[0], sem)

            def chunk(j, carry):
                r = lax.rem(j, 3)

                @pl.when(j >= 2)
                def _():
                    pltpu.make_async_copy(rows_v.at[0],
                                          acc.at[dst_r.at[0]], ssem).wait()

                @pl.when(j + 2 < rpt)
                def _():
                    pltpu.async_copy(src_hbm.at[base + j + 2],
                                     src_r.at[lax.rem(j + 2, 3)], isem)
                    pltpu.async_copy(dst_hbm.at[base + j + 2],
                                     dst_r.at[lax.rem(j + 2, 4)], dsem)

                @pl.when(j + 1 < rpt)
                def _():
                    pltpu.make_async_copy(src_hbm.at[base + j + 1],
                                          src_r.at[0], isem).wait()
                    nr = lax.rem(j + 1, 3)
                    pltpu.async_copy(tbl.at[src_r.at[nr]],
                                     rows_v.at[nr], sem)

                pltpu.make_async_copy(tbl.at[src_r.at[0]],
                                      rows_v.at[r], sem).wait()
                pltpu.make_async_copy(dst_hbm.at[base + j],
                                      dst_r.at[0], dsem).wait()
                pltpu.async_copy(rows_v.at[r], acc.at[dst_r.at[lax.rem(j, 4)]],
                                 ssem, add=True)
                return carry

            lax.fori_loop(0, rpt, chunk, 0)
            for _ in range(2):
                pltpu.make_async_copy(rows_v.at[0], acc.at[dst_r.at[0]],
                                      ssem).wait()

        @pl.when(cid == 0)
        def _():
            run(y0_hbm)

        @pl.when(cid == 1)
        def _():
            run(y1_hbm)

        plsc.subcore_barrier()
        pltpu.sync_copy(acc.at[pl.ds(sid * spt, spt)],
                        out_hbm.at[cid, pl.ds(sid * spt, spt)])

    zrows = jnp.zeros((spt, h2), jnp.float32)
    fn = pl.kernel(
        body,
        out_type=jax.ShapeDtypeStruct((_NC, nacc, h2), jnp.float32),
        mesh=mesh,
        scratch_types=[
            pltpu.VMEM((3, _CH), jnp.int32),
            pltpu.VMEM((4, _CH), jnp.int32),
            pltpu.VMEM((3, _CH, h2), jnp.float32),
            pltpu.VMEM_SHARED((nacc, h2), jnp.float32),
            pltpu.SemaphoreType.DMA,
            pltpu.SemaphoreType.DMA,
            pltpu.SemaphoreType.DMA,
            pltpu.SemaphoreType.DMA,
        ],
        interpret=interpret,
    )
    return fn(src2d, dst2d, y0, y1, zrows)


def _dinv_from(deg_ref, blk):
    deg = deg_ref[0][:, 0] + deg_ref[1][:, 0] + 1.0
    return lax.rsqrt(deg)


def _tc_matmul_plain(x, w, nblk, blk, interpret=False):
    n, d = x.shape
    h = w.shape[1]

    def body(x_ref, w_ref, y_ref):
        y_ref[...] = jnp.dot(x_ref[...], w_ref[...],
                             preferred_element_type=jnp.float32)

    return pl.pallas_call(
        body,
        grid=(nblk,),
        in_specs=[
            pl.BlockSpec((blk, d), lambda i: (i, 0)),
            pl.BlockSpec((d, h), lambda i: (0, 0)),
        ],
        out_specs=pl.BlockSpec((blk, h), lambda i: (i, 0)),
        out_shape=jax.ShapeDtypeStruct((n, h), jnp.float32),
        interpret=interpret,
    )(x, w)


def _tc_scale_split(xw, degp, nblk, blk, interpret=False):
    n, h = xw.shape
    h2 = h // 2

    def body(x_ref, deg_ref, y_ref):
        dinv = _dinv_from(deg_ref, blk)
        y = x_ref[...] * dinv[:, None]
        y_ref[0] = y[:, :h2]
        y_ref[1] = y[:, h2:]

    return pl.pallas_call(
        body,
        grid=(nblk,),
        in_specs=[
            pl.BlockSpec((blk, h), lambda i: (i, 0)),
            pl.BlockSpec((2, blk, _DW), lambda i: (0, i, 0)),
        ],
        out_specs=pl.BlockSpec((2, blk, h2), lambda i: (0, i, 0)),
        out_shape=jax.ShapeDtypeStruct((2, n, h2), jnp.float32),
        interpret=interpret,
    )(xw, degp)


def _tc_conv_stats(S, y, degp, bias2, nblk, blk, interpret=False):
    n = y.shape[1]
    h2 = y.shape[2]

    def body(s_ref, y_ref, deg_ref, b_ref, conv_ref, st_ref):
        i = pl.program_id(0)
        dinv = _dinv_from(deg_ref, blk)

        @pl.when(i == 0)
        def _():
            st_ref[...] = jnp.zeros_like(st_ref)

        for c in range(2):
            conv = (s_ref[c] + y_ref[c]) * dinv[:, None] + b_ref[c]
            conv_ref[c] = conv
            upd = jnp.concatenate(
                [jnp.sum(conv, axis=0)[None, :],
                 jnp.sum(conv * conv, axis=0)[None, :],
                 jnp.zeros((6, h2), jnp.float32)], axis=0)
            st_ref[c] = st_ref[c] + upd

    return pl.pallas_call(
        body,
        grid=(nblk,),
        in_specs=[
            pl.BlockSpec((2, blk, h2), lambda i: (0, i, 0)),
            pl.BlockSpec((2, blk, h2), lambda i: (0, i, 0)),
            pl.BlockSpec((2, blk, _DW), lambda i: (0, i, 0)),
            pl.BlockSpec((2, 1, h2), lambda i: (0, 0, 0)),
        ],
        out_specs=[
            pl.BlockSpec((2, blk, h2), lambda i: (0, i, 0)),
            pl.BlockSpec((2, 8, h2), lambda i: (0, 0, 0)),
        ],
        out_shape=[
            jax.ShapeDtypeStruct((2, n, h2), jnp.float32),
            jax.ShapeDtypeStruct((2, 8, h2), jnp.float32),
        ],
        interpret=interpret,
    )(S, y, degp, bias2)


def _bn_relu_halves(conv_ref, st_ref, g_ref, be_ref, n_nodes, h2):
    hs = []
    for c in range(2):
        mean = st_ref[c, 0:1, :] / n_nodes
        var = st_ref[c, 1:2, :] / n_nodes - mean * mean
        rstd = lax.rsqrt(var + _EPS)
        xn = (conv_ref[c] - mean) * rstd
        hs.append(jnp.maximum(xn * g_ref[c] + be_ref[c], 0.0))
    return jnp.concatenate(hs, axis=1)


def _tc_bn_matmul_scaled(conv, st, gam, bet, w, degp, n_nodes, nblk, blk,
                         interpret=False):
    n = conv.shape[1]
    h2 = conv.shape[2]

    def body(conv_ref, st_ref, g_ref, be_ref, w_ref, deg_ref, y_ref):
        dinv = _dinv_from(deg_ref, blk)
        hcat = _bn_relu_halves(conv_ref, st_ref, g_ref, be_ref, n_nodes, h2)
        y = jnp.dot(hcat, w_ref[...],
                    preferred_element_type=jnp.float32) * dinv[:, None]
        y_ref[0] = y[:, :h2]
        y_ref[1] = y[:, h2:]

    return pl.pallas_call(
        body,
        grid=(nblk,),
        in_specs=[
            pl.BlockSpec((2, blk, h2), lambda i: (0, i, 0)),
            pl.BlockSpec((2, 8, h2), lambda i: (0, 0, 0)),
            pl.BlockSpec((2, 1, h2), lambda i: (0, 0, 0)),
            pl.BlockSpec((2, 1, h2), lambda i: (0, 0, 0)),
            pl.BlockSpec((2 * h2, 2 * h2), lambda i: (0, 0)),
            pl.BlockSpec((2, blk, _DW), lambda i: (0, i, 0)),
        ],
        out_specs=pl.BlockSpec((2, blk, h2), lambda i: (0, i, 0)),
        out_shape=jax.ShapeDtypeStruct((2, n, h2), jnp.float32),
        interpret=interpret,
    )(conv, st, gam, bet, w, degp)


def _tc_bn_pool_head(conv, st, gam, bet, batch3, whp, bhp, n_nodes, n_graphs,
                     nblk, blk, interpret=False):
    h2 = conv.shape[2]
    hp = whp.shape[1]

    def body(conv_ref, st_ref, g_ref, be_ref, b3_ref, wh_ref, bh_ref,
             out_ref, psum, cnt):
        i = pl.program_id(0)
        hcat = _bn_relu_halves(conv_ref, st_ref, g_ref, be_ref, n_nodes, h2)
        bb = b3_ref[0, 0, :]
        oh = (bb[:, None] == lax.broadcasted_iota(
            jnp.int32, (blk, n_graphs), 1)).astype(jnp.float32)

        @pl.when(i == 0)
        def _():
            psum[...] = jnp.zeros_like(psum)
            cnt[...] = jnp.zeros_like(cnt)

        dn = (((0,), (0,)), ((), ()))
        psum[...] += lax.dot_general(oh, hcat, dn,
                                     preferred_element_type=jnp.float32)
        cnt[...] += lax.dot_general(oh, jnp.ones((blk, hp), jnp.float32), dn,
                                    preferred_element_type=jnp.float32)

        @pl.when(i == nblk - 1)
        def _():
            pooled = psum[...] / jnp.maximum(cnt[...][:, 0:1], 1.0)
            out_ref[...] = jnp.dot(pooled, wh_ref[...],
                                   preferred_element_type=jnp.float32) + bh_ref[...]

    return pl.pallas_call(
        body,
        grid=(nblk,),
        in_specs=[
            pl.BlockSpec((2, blk, h2), lambda i: (0, i, 0)),
            pl.BlockSpec((2, 8, h2), lambda i: (0, 0, 0)),
            pl.BlockSpec((2, 1, h2), lambda i: (0, 0, 0)),
            pl.BlockSpec((2, 1, h2), lambda i: (0, 0, 0)),
            pl.BlockSpec((1, 1, blk), lambda i: (i, 0, 0)),
            pl.BlockSpec((2 * h2, hp), lambda i: (0, 0)),
            pl.BlockSpec((1, hp), lambda i: (0, 0)),
        ],
        out_specs=pl.BlockSpec((n_graphs, hp), lambda i: (0, 0)),
        out_shape=jax.ShapeDtypeStruct((n_graphs, hp), jnp.float32),
        scratch_shapes=[
            pltpu.VMEM((n_graphs, 2 * h2), jnp.float32),
            pltpu.VMEM((n_graphs, hp), jnp.float32),
        ],
        interpret=interpret,
    )(conv, st, gam, bet, batch3, whp, bhp)


def kernel(graph_x, edge_index, batch, W1, b1, gamma1, beta1,
           W2, b2, gamma2, beta2, Wh, bh):
    n, d = graph_x.shape
    e = edge_index.shape[1]
    h = W1.shape[1]
    h2 = h // 2
    o = Wh.shape[1]
    n_graphs = 64
    blk = 400
    nblk = n // blk

    rows = _round_up(e, _CH * _NC * _NS) // _CH
    pad = rows * _CH - e
    nacc = _round_up(n + 1, 128)

    src = edge_index[0]
    dst = edge_index[1]
    src2d = jnp.concatenate(
        [src, jnp.zeros((pad,), src.dtype)]).reshape(rows, _CH)
    dst2d = jnp.concatenate(
        [dst, jnp.full((pad,), n, dst.dtype)]).reshape(rows, _CH)

    b1h = b1.reshape(2, 1, h2)
    g1h = gamma1.reshape(2, 1, h2)
    be1h = beta1.reshape(2, 1, h2)
    b2h = b2.reshape(2, 1, h2)
    g2h = gamma2.reshape(2, 1, h2)
    be2h = beta2.reshape(2, 1, h2)

    degp = _sc_degree(dst2d, nacc)

    xw1 = _tc_matmul_plain(graph_x.astype(jnp.float32), W1, nblk, blk)
    y1 = _tc_scale_split(xw1, degp, nblk, blk)
    S1 = _sc_aggregate(src2d, dst2d, y1[0], y1[1], nacc)
    c1, st1 = _tc_conv_stats(S1, y1, degp, b1h, nblk, blk)

    y2 = _tc_bn_matmul_scaled(c1, st1, g1h, be1h, W2, degp, float(n),
                              nblk, blk)
    S2 = _sc_aggregate(src2d, dst2d, y2[0], y2[1], nacc)
    c2, st2 = _tc_conv_stats(S2, y2, degp, b2h, nblk, blk)

    batch3 = batch.reshape(nblk, 1, blk)
    hp = 128
    whp = jnp.zeros((h, hp), jnp.float32).at[:, :o].set(Wh)
    bhp = jnp.zeros((1, hp), jnp.float32).at[0, :o].set(bh)
    outp = _tc_bn_pool_head(c2, st2, g2h, be2h, batch3, whp, bhp,
                            float(n), n_graphs, nblk, blk)
    return outp[:, :o]
